# baseline (speedup 1.0000x reference)
import functools

import jax
import jax.numpy as jnp
from jax import lax
from jax.experimental import pallas as pl
from jax.experimental.pallas import tpu as pltpu
from jax.experimental.pallas import tpu_sc as plsc

N = 10000
E = 160000
D = 256
H = 512
L = 128
G = 256
T = 3

NB = 400
NBLK = N // NB
HPAD = 26 * NB
HALF_T = 5200
ACC2 = 5248
DUMMY_L = 5216
AGG_B1 = 5600
AGG_R = AGG_B1 + ACC2
ZR = ACC2 // 16
E_P = 163840
SLICE_R = 80
SLOT = 176
RB = 16

_mesh = plsc.VectorSubcoreMesh(core_axis_name="c", subcore_axis_name="s")


@functools.partial(
    pl.kernel,
    out_type=[
        jax.ShapeDtypeStruct((32 * SLOT, 64), jnp.int32),
        jax.ShapeDtypeStruct((32 * SLOT, 64), jnp.int32),
        jax.ShapeDtypeStruct((512,), jnp.int32),
    ],
    mesh=_mesh,
    compiler_params=pltpu.CompilerParams(needs_layout_passes=False),
    scratch_types=[
        pltpu.VMEM((SLICE_R, 128), jnp.int32),
        pltpu.VMEM((SLICE_R, 128), jnp.int32),
        pltpu.VMEM((SLOT, 64), jnp.int32),
        pltpu.VMEM((SLOT, 64), jnp.int32),
        pltpu.VMEM((192,), jnp.int32),
        pltpu.VMEM((192,), jnp.int32),
        pltpu.VMEM((16,), jnp.int32),
    ],
)
def _sc_partition(src_hbm, dst_hbm, slists_hbm, dlists_hbm, counts_hbm,
                  src_st, dst_st, s_list, d_list, srow, drow, cbuf):
    c = lax.axis_index("c")
    s = lax.axis_index("s")
    wid = c * 16 + s
    pltpu.sync_copy(src_hbm.at[pl.ds(s * SLICE_R, SLICE_R)], src_st)
    pltpu.sync_copy(dst_hbm.at[pl.ds(s * SLICE_R, SLICE_R)], dst_st)
    lo = c * HALF_T

    def flush(nrow):
        for k in range(4):
            sl = pl.ds(16 * k, 16)
            s_list[nrow, sl] = srow[sl]
            d_list[nrow, sl] = drow[sl]

    trash = jnp.arange(16, dtype=jnp.int32) + 176
    ones = jnp.ones((16,), jnp.int32)
    zeros = jnp.zeros((16,), jnp.int32)

    def row_loop(r, carry):
        wp, nrow = carry
        for k in range(8):
            sl = pl.ds(16 * k, 16)
            sv = src_st[r, sl]
            dv = dst_st[r, sl]
            t = dv - lo
            m = jnp.logical_and(t >= 0, t < HALF_T)
            mi = jnp.where(m, ones, zeros)
            offs = jnp.where(m, wp + jnp.cumsum(mi) - 1, trash)
            plsc.store_scatter(srow, [offs], sv)
            plsc.store_scatter(drow, [offs], t)
            wp = wp + jnp.sum(mi)
            full = wp >= 64

            @pl.when(full)
            def _():
                flush(nrow)
                srow[pl.ds(0, 16)] = srow[pl.ds(64, 16)]
                drow[pl.ds(0, 16)] = drow[pl.ds(64, 16)]

            wp = jnp.where(full, wp - 64, wp)
            nrow = nrow + full.astype(jnp.int32)
        return (wp, nrow)

    wp, nrow = lax.fori_loop(0, SLICE_R, row_loop,
                             (jnp.int32(0), jnp.int32(0)))

    dummy_s = jnp.zeros((16,), jnp.int32)
    dummy_d = jnp.full((16,), DUMMY_L, jnp.int32)
    lane = jnp.arange(16, dtype=jnp.int32)
    for k in range(4):
        offs = wp + lane + 16 * k
        plsc.store_scatter(srow, [offs], dummy_s)
        plsc.store_scatter(drow, [offs], dummy_d)
    flush(nrow)
    for k in range(4):
        sl = pl.ds(16 * k, 16)
        srow[sl] = dummy_s
        drow[sl] = dummy_d
    flush(nrow + 1)
    odd = lax.rem(nrow, 2) == 1

    @pl.when(odd)
    def _():
        flush(nrow + 2)

    nsub_out = nrow + 2 + jnp.where(odd, 1, 0)
    cbuf[pl.ds(0, 16)] = jnp.full((16,), nsub_out, jnp.int32)
    pltpu.sync_copy(s_list, slists_hbm.at[pl.ds(wid * SLOT, SLOT)])
    pltpu.sync_copy(d_list, dlists_hbm.at[pl.ds(wid * SLOT, SLOT)])
    pltpu.sync_copy(cbuf, counts_hbm.at[pl.ds(wid * 16, 16)])


@functools.partial(
    pl.kernel,
    out_type=jax.ShapeDtypeStruct((AGG_R, 2, 128), jnp.float32),
    mesh=_mesh,
    scratch_types=[
        pltpu.VMEM((32, 64), jnp.int32),
        pltpu.VMEM((RB, 64), jnp.int32),
        pltpu.VMEM((64, 2, 128), jnp.float32),
        pltpu.VMEM((64, 2, 128), jnp.float32),
        pltpu.VMEM((16,), jnp.int32),
        pltpu.VMEM_SHARED((ACC2, 2, 128), jnp.float32),
        pltpu.SemaphoreType.DMA,
        pltpu.SemaphoreType.DMA,
    ],
)
def _sc_gather_scatter(h_hbm, slists_hbm, dlists_hbm, counts_hbm, agg_hbm,
                       sidx, didx, bufa, bufb, cbuf, acc, sema, semb):
    c = lax.axis_index("c")
    s = lax.axis_index("s")
    wid = c * 16 + s
    slotr = wid * SLOT

    pltpu.sync_copy(counts_hbm.at[pl.ds(wid * 16, 16)], cbuf)
    nsub = cbuf[...][0]

    pltpu.sync_copy(slists_hbm.at[pl.ds(slotr, 32)], sidx)
    pltpu.sync_copy(dlists_hbm.at[pl.ds(slotr, RB)], didx)

    zero = jnp.zeros((16,), jnp.float32)

    def zrow(i, carry):
        for half in range(2):
            for k in range(128 // 16):
                bufa[i, half, pl.ds(k * 16, 16)] = zero
        return carry

    lax.fori_loop(0, 64, zrow, 0)
    base = s * ZR
    for k in range(ZR // 64):
        pltpu.sync_copy(bufa, acc.at[pl.ds(base + k * 64, 64)])
    rem = ZR % 64
    if rem:
        pltpu.sync_copy(bufa.at[pl.ds(0, rem)],
                        acc.at[pl.ds(base + (ZR // 64) * 64, rem)])
    plsc.subcore_barrier()

    def g_start(g, buf, sem):
        pltpu.make_async_copy(h_hbm.at[sidx.at[lax.rem(g, 32)]], buf,
                              sem).start()

    def g_wait(g, buf, sem):
        pltpu.make_async_copy(h_hbm.at[sidx.at[lax.rem(g, 32)]], buf,
                              sem).wait()

    g_start(0, bufa, sema)
    g_start(1, bufb, semb)

    def pair(p, carry):
        g0 = 2 * p
        g1 = g0 + 1

        at_block = jnp.logical_and(g0 > 0, lax.rem(g0, RB) == 0)

        @pl.when(at_block)
        def _():
            b = (g0 // RB) * RB
            pltpu.sync_copy(dlists_hbm.at[pl.ds(slotr + b, RB)], didx)

        @pl.when(jnp.logical_and(at_block, g0 + 2 * RB <= SLOT))
        def _():
            b = (g0 // RB) * RB
            so = lax.rem(b // RB + 1, 2) * RB
            pltpu.sync_copy(slists_hbm.at[pl.ds(slotr + b + RB, RB)],
                            sidx.at[pl.ds(so, RB)])

        g_wait(g0, bufa, sema)
        pltpu.sync_copy(bufa, acc.at[didx.at[lax.rem(g0, RB)]], add=True)

        @pl.when(g0 + 2 < nsub)
        def _():
            g_start(g0 + 2, bufa, sema)

        g_wait(g1, bufb, semb)
        pltpu.sync_copy(bufb, acc.at[didx.at[lax.rem(g1, RB)]], add=True)

        @pl.when(g1 + 2 < nsub)
        def _():
            g_start(g1 + 2, bufb, semb)

        return carry

    lax.fori_loop(0, nsub // 2, pair, 0)
    plsc.subcore_barrier()

    outb = c * AGG_B1 + s * ZR
    for k in range(ZR // 64):
        pltpu.sync_copy(acc.at[pl.ds(base + k * 64, 64)],
                        agg_hbm.at[pl.ds(outb + k * 64, 64)])
    if rem:
        pltpu.sync_copy(acc.at[pl.ds(base + (ZR // 64) * 64, rem)],
                        agg_hbm.at[pl.ds(outb + (ZR // 64) * 64, rem)])


def _tc_in_body(x_ref, w_ref, o_ref):
    o_ref[...] = jnp.maximum(
        jnp.dot(x_ref[...], w_ref[...], preferred_element_type=jnp.float32), 0.0)


_tc_in = pl.pallas_call(
    _tc_in_body,
    grid=(NBLK,),
    in_specs=[
        pl.BlockSpec((NB, D), lambda i: (i, 0)),
        pl.BlockSpec((D, D), lambda i: (0, 0)),
    ],
    out_specs=pl.BlockSpec((NB, D), lambda i: (i, 0)),
    out_shape=jax.ShapeDtypeStruct((HPAD, D), jnp.float32),
)


def _tc_step_body(h_ref, a_ref, w_ref, o_ref):
    o_ref[...] = jnp.maximum(
        h_ref[...] + jnp.dot(a_ref[...], w_ref[...],
                             preferred_element_type=jnp.float32), 0.0)


_tc_step = pl.pallas_call(
    _tc_step_body,
    grid=(NBLK,),
    in_specs=[
        pl.BlockSpec((NB, D), lambda i: (i, 0)),
        pl.BlockSpec((NB, D), lambda i: (jnp.where(i < HALF_T // NB, i, i + 1), 0)),
        pl.BlockSpec((D, D), lambda i: (0, 0)),
    ],
    out_specs=pl.BlockSpec((NB, D), lambda i: (i, 0)),
    out_shape=jax.ShapeDtypeStruct((HPAD, D), jnp.float32),
)


def _tc_head_body(h_ref, gid_ref, wf_ref, bf_ref, wm_ref, bm_ref,
                  wl_ref, bl_ref, mu_ref, lv_ref, g_acc, c_acc):
    i = pl.program_id(0)

    @pl.when(i == 0)
    def _():
        g_acc[...] = jnp.zeros_like(g_acc)
        c_acc[...] = jnp.zeros_like(c_acc)

    gid = gid_ref[0]
    mt = (lax.broadcasted_iota(jnp.int32, (G, NB), 0) == gid).astype(jnp.float32)
    g_acc[...] += jnp.dot(mt, h_ref[...], preferred_element_type=jnp.float32)
    c_acc[...] += jnp.sum(mt, axis=1, keepdims=True)

    @pl.when(i == NBLK - 1)
    def _():
        cnt = jnp.maximum(c_acc[...], 1.0)
        g = g_acc[...] / cnt
        hh = jnp.maximum(
            jnp.dot(g, wf_ref[...], preferred_element_type=jnp.float32)
            + bf_ref[...], 0.0)
        mu_ref[...] = jnp.dot(hh, wm_ref[...],
                              preferred_element_type=jnp.float32) + bm_ref[...]
        lv_ref[...] = jnp.dot(hh, wl_ref[...],
                              preferred_element_type=jnp.float32) + bl_ref[...]


_tc_head = pl.pallas_call(
    _tc_head_body,
    grid=(NBLK,),
    in_specs=[
        pl.BlockSpec((NB, D), lambda i: (i, 0)),
        pl.BlockSpec((1, 1, NB), lambda i: (i, 0, 0)),
        pl.BlockSpec((D, H), lambda i: (0, 0)),
        pl.BlockSpec((1, H), lambda i: (0, 0)),
        pl.BlockSpec((H, L), lambda i: (0, 0)),
        pl.BlockSpec((1, L), lambda i: (0, 0)),
        pl.BlockSpec((H, L), lambda i: (0, 0)),
        pl.BlockSpec((1, L), lambda i: (0, 0)),
    ],
    out_specs=[
        pl.BlockSpec((G, L), lambda i: (0, 0)),
        pl.BlockSpec((G, L), lambda i: (0, 0)),
    ],
    out_shape=[
        jax.ShapeDtypeStruct((G, L), jnp.float32),
        jax.ShapeDtypeStruct((G, L), jnp.float32),
    ],
    scratch_shapes=[
        pltpu.VMEM((G, D), jnp.float32),
        pltpu.VMEM((G, 1), jnp.float32),
    ],
)


@jax.jit
def kernel(x, edge_index, graph_ids, W_in, W_msg, W_fc1, b_fc1, W_mu, b_mu,
           W_lv, b_lv):
    src = edge_index[0]
    dst = edge_index[1]
    pad = E_P - E
    srcp = jnp.concatenate([src, jnp.zeros((pad,), jnp.int32)]
                           ).reshape(E_P // 128, 128)
    dstp = jnp.concatenate([dst, jnp.full((pad,), 2 * HALF_T, jnp.int32)]
                           ).reshape(E_P // 128, 128)
    gidp = graph_ids.reshape(NBLK, 1, NB)
    bf = b_fc1.reshape(1, H)
    bm = b_mu.reshape(1, L)
    bl = b_lv.reshape(1, L)

    slists, dlists, counts = _sc_partition(srcp, dstp)
    h = _tc_in(x, W_in)
    for _ in range(T):
        agg = _sc_gather_scatter(h.reshape(HPAD, 2, 128), slists, dlists,
                                 counts)
        h = _tc_step(h, agg.reshape(AGG_R, D), W_msg)
    mu, lv = _tc_head(h, gidp, W_fc1, bf, W_mu, bm, W_lv, bl)
    return (mu, lv)


# speedup vs baseline: 1.1053x; 1.1053x over previous
import functools

import jax
import jax.numpy as jnp
from jax import lax
from jax.experimental import pallas as pl
from jax.experimental.pallas import tpu as pltpu
from jax.experimental.pallas import tpu_sc as plsc

N = 10000
E = 160000
D = 256
H = 512
L = 128
G = 256
T = 3

NB = 400
NBLK = N // NB
HPAD = 26 * NB
HALF_T = 5200
ACC2 = 5248
DUMMY_L = 5216
AGG_B1 = 5600
AGG_R = AGG_B1 + ACC2
ZR = ACC2 // 16
E_P = 163840
SLICE_R = 80
SLOT = 88
RB = 16

_mesh = plsc.VectorSubcoreMesh(core_axis_name="c", subcore_axis_name="s")


@functools.partial(
    pl.kernel,
    out_type=[
        jax.ShapeDtypeStruct((32 * SLOT, 128), jnp.int32),
        jax.ShapeDtypeStruct((32 * SLOT, 128), jnp.int32),
        jax.ShapeDtypeStruct((512,), jnp.int32),
    ],
    mesh=_mesh,
    compiler_params=pltpu.CompilerParams(needs_layout_passes=False),
    scratch_types=[
        pltpu.VMEM((SLICE_R, 128), jnp.int32),
        pltpu.VMEM((SLICE_R, 128), jnp.int32),
        pltpu.VMEM((SLOT, 128), jnp.int32),
        pltpu.VMEM((SLOT, 128), jnp.int32),
        pltpu.VMEM((256,), jnp.int32),
        pltpu.VMEM((256,), jnp.int32),
        pltpu.VMEM((16,), jnp.int32),
    ],
)
def _sc_partition(src_hbm, dst_hbm, slists_hbm, dlists_hbm, counts_hbm,
                  src_st, dst_st, s_list, d_list, srow, drow, cbuf):
    c = lax.axis_index("c")
    s = lax.axis_index("s")
    wid = c * 16 + s
    pltpu.sync_copy(src_hbm.at[pl.ds(s * SLICE_R, SLICE_R)], src_st)
    pltpu.sync_copy(dst_hbm.at[pl.ds(s * SLICE_R, SLICE_R)], dst_st)
    lo = c * HALF_T

    def flush(nrow):
        for k in range(8):
            sl = pl.ds(16 * k, 16)
            s_list[nrow, sl] = srow[sl]
            d_list[nrow, sl] = drow[sl]

    trash = jnp.arange(16, dtype=jnp.int32) + 240
    ones = jnp.ones((16,), jnp.int32)
    zeros = jnp.zeros((16,), jnp.int32)

    def row_loop(r, carry):
        wp, nrow = carry
        for k in range(8):
            sl = pl.ds(16 * k, 16)
            sv = src_st[r, sl]
            dv = dst_st[r, sl]
            t = dv - lo
            m = jnp.logical_and(t >= 0, t < HALF_T)
            mi = jnp.where(m, ones, zeros)
            offs = jnp.where(m, wp + jnp.cumsum(mi) - 1, trash)
            plsc.store_scatter(srow, [offs], sv)
            plsc.store_scatter(drow, [offs], t)
            wp = wp + jnp.sum(mi)
            full = wp >= 128

            @pl.when(full)
            def _():
                flush(nrow)
                srow[pl.ds(0, 16)] = srow[pl.ds(128, 16)]
                drow[pl.ds(0, 16)] = drow[pl.ds(128, 16)]

            wp = jnp.where(full, wp - 128, wp)
            nrow = nrow + full.astype(jnp.int32)
        return (wp, nrow)

    wp, nrow = lax.fori_loop(0, SLICE_R, row_loop,
                             (jnp.int32(0), jnp.int32(0)))

    dummy_s = jnp.zeros((16,), jnp.int32)
    dummy_d = jnp.full((16,), DUMMY_L, jnp.int32)
    lane = jnp.arange(16, dtype=jnp.int32)
    for k in range(8):
        offs = wp + lane + 16 * k
        plsc.store_scatter(srow, [offs], dummy_s)
        plsc.store_scatter(drow, [offs], dummy_d)
    flush(nrow)

    cbuf[pl.ds(0, 16)] = jnp.full((16,), nrow + 1, jnp.int32)
    pltpu.sync_copy(s_list, slists_hbm.at[pl.ds(wid * SLOT, SLOT)])
    pltpu.sync_copy(d_list, dlists_hbm.at[pl.ds(wid * SLOT, SLOT)])
    pltpu.sync_copy(cbuf, counts_hbm.at[pl.ds(wid * 16, 16)])


@functools.partial(
    pl.kernel,
    out_type=jax.ShapeDtypeStruct((AGG_R, 2, 128), jnp.float32),
    mesh=_mesh,
    scratch_types=[
        pltpu.VMEM((RB, 128), jnp.int32),
        pltpu.VMEM((RB, 128), jnp.int32),
        pltpu.VMEM((128, 2, 128), jnp.float32),
        pltpu.VMEM((16,), jnp.int32),
        pltpu.VMEM_SHARED((ACC2, 2, 128), jnp.float32),
        pltpu.SemaphoreType.DMA,
    ],
)
def _sc_gather_scatter(h_hbm, slists_hbm, dlists_hbm, counts_hbm, agg_hbm,
                       sidx, didx, buf, cbuf, acc, sem):
    c = lax.axis_index("c")
    s = lax.axis_index("s")
    wid = c * 16 + s
    slotr = wid * SLOT

    pltpu.sync_copy(counts_hbm.at[pl.ds(wid * 16, 16)], cbuf)
    nsub = cbuf[...][0]

    pltpu.sync_copy(slists_hbm.at[pl.ds(slotr, RB)], sidx)
    pltpu.sync_copy(dlists_hbm.at[pl.ds(slotr, RB)], didx)

    zero = jnp.zeros((16,), jnp.float32)

    def zrow(i, carry):
        for half in range(2):
            for k in range(128 // 16):
                buf[i, half, pl.ds(k * 16, 16)] = zero
        return carry

    lax.fori_loop(0, 128, zrow, 0)
    base = s * ZR
    for k in range(ZR // 128):
        pltpu.sync_copy(buf, acc.at[pl.ds(base + k * 128, 128)])
    rem = ZR % 128
    if rem:
        pltpu.sync_copy(buf.at[pl.ds(0, rem)],
                        acc.at[pl.ds(base + (ZR // 128) * 128, rem)])
    plsc.subcore_barrier()

    def body(g, carry):
        @pl.when(jnp.logical_and(g > 0, lax.rem(g, RB) == 0))
        def _():
            b = (g // RB) * RB
            pltpu.sync_copy(slists_hbm.at[pl.ds(slotr + b, RB)], sidx)
            pltpu.sync_copy(dlists_hbm.at[pl.ds(slotr + b, RB)], didx)

        r = lax.rem(g, RB)
        pltpu.make_async_copy(h_hbm.at[sidx.at[r]], buf, sem).start()
        pltpu.make_async_copy(h_hbm.at[sidx.at[r]], buf, sem).wait()
        pltpu.sync_copy(buf, acc.at[didx.at[r]], add=True)
        return carry

    lax.fori_loop(0, nsub, body, 0)
    plsc.subcore_barrier()

    outb = c * AGG_B1 + s * ZR
    for k in range(ZR // 128):
        pltpu.sync_copy(acc.at[pl.ds(base + k * 128, 128)],
                        agg_hbm.at[pl.ds(outb + k * 128, 128)])
    if rem:
        pltpu.sync_copy(acc.at[pl.ds(base + (ZR // 128) * 128, rem)],
                        agg_hbm.at[pl.ds(outb + (ZR // 128) * 128, rem)])


def _tc_in_body(x_ref, w_ref, o_ref):
    o_ref[...] = jnp.maximum(
        jnp.dot(x_ref[...], w_ref[...], preferred_element_type=jnp.float32), 0.0)


_tc_in = pl.pallas_call(
    _tc_in_body,
    grid=(NBLK,),
    in_specs=[
        pl.BlockSpec((NB, D), lambda i: (i, 0)),
        pl.BlockSpec((D, D), lambda i: (0, 0)),
    ],
    out_specs=pl.BlockSpec((NB, D), lambda i: (i, 0)),
    out_shape=jax.ShapeDtypeStruct((HPAD, D), jnp.float32),
)


def _tc_step_body(h_ref, a_ref, w_ref, o_ref):
    o_ref[...] = jnp.maximum(
        h_ref[...] + jnp.dot(a_ref[...], w_ref[...],
                             preferred_element_type=jnp.float32), 0.0)


_tc_step = pl.pallas_call(
    _tc_step_body,
    grid=(NBLK,),
    in_specs=[
        pl.BlockSpec((NB, D), lambda i: (i, 0)),
        pl.BlockSpec((NB, D), lambda i: (jnp.where(i < HALF_T // NB, i, i + 1), 0)),
        pl.BlockSpec((D, D), lambda i: (0, 0)),
    ],
    out_specs=pl.BlockSpec((NB, D), lambda i: (i, 0)),
    out_shape=jax.ShapeDtypeStruct((HPAD, D), jnp.float32),
)


def _tc_head_body(h_ref, a_ref, wg_ref, gid_ref, wf_ref, bf_ref, wm_ref,
                  bm_ref, wl_ref, bl_ref, mu_ref, lv_ref, g_acc, c_acc):
    i = pl.program_id(0)

    @pl.when(i == 0)
    def _():
        g_acc[...] = jnp.zeros_like(g_acc)
        c_acc[...] = jnp.zeros_like(c_acc)

    hblk = jnp.maximum(
        h_ref[...] + jnp.dot(a_ref[...], wg_ref[...],
                             preferred_element_type=jnp.float32), 0.0)
    gid = gid_ref[0]
    mt = (lax.broadcasted_iota(jnp.int32, (G, NB), 0) == gid).astype(jnp.float32)
    g_acc[...] += jnp.dot(mt, hblk, preferred_element_type=jnp.float32)
    c_acc[...] += jnp.sum(mt, axis=1, keepdims=True)

    @pl.when(i == NBLK - 1)
    def _():
        cnt = jnp.maximum(c_acc[...], 1.0)
        g = g_acc[...] / cnt
        hh = jnp.maximum(
            jnp.dot(g, wf_ref[...], preferred_element_type=jnp.float32)
            + bf_ref[...], 0.0)
        mu_ref[...] = jnp.dot(hh, wm_ref[...],
                              preferred_element_type=jnp.float32) + bm_ref[...]
        lv_ref[...] = jnp.dot(hh, wl_ref[...],
                              preferred_element_type=jnp.float32) + bl_ref[...]


_tc_head = pl.pallas_call(
    _tc_head_body,
    grid=(NBLK,),
    in_specs=[
        pl.BlockSpec((NB, D), lambda i: (i, 0)),
        pl.BlockSpec((NB, D), lambda i: (jnp.where(i < HALF_T // NB, i, i + 1), 0)),
        pl.BlockSpec((D, D), lambda i: (0, 0)),
        pl.BlockSpec((1, 1, NB), lambda i: (i, 0, 0)),
        pl.BlockSpec((D, H), lambda i: (0, 0)),
        pl.BlockSpec((1, H), lambda i: (0, 0)),
        pl.BlockSpec((H, L), lambda i: (0, 0)),
        pl.BlockSpec((1, L), lambda i: (0, 0)),
        pl.BlockSpec((H, L), lambda i: (0, 0)),
        pl.BlockSpec((1, L), lambda i: (0, 0)),
    ],
    out_specs=[
        pl.BlockSpec((G, L), lambda i: (0, 0)),
        pl.BlockSpec((G, L), lambda i: (0, 0)),
    ],
    out_shape=[
        jax.ShapeDtypeStruct((G, L), jnp.float32),
        jax.ShapeDtypeStruct((G, L), jnp.float32),
    ],
    scratch_shapes=[
        pltpu.VMEM((G, D), jnp.float32),
        pltpu.VMEM((G, 1), jnp.float32),
    ],
)


@jax.jit
def kernel(x, edge_index, graph_ids, W_in, W_msg, W_fc1, b_fc1, W_mu, b_mu,
           W_lv, b_lv):
    src = edge_index[0]
    dst = edge_index[1]
    pad = E_P - E
    srcp = jnp.concatenate([src, jnp.zeros((pad,), jnp.int32)]
                           ).reshape(E_P // 128, 128)
    dstp = jnp.concatenate([dst, jnp.full((pad,), 2 * HALF_T, jnp.int32)]
                           ).reshape(E_P // 128, 128)
    gidp = graph_ids.reshape(NBLK, 1, NB)
    bf = b_fc1.reshape(1, H)
    bm = b_mu.reshape(1, L)
    bl = b_lv.reshape(1, L)

    slists, dlists, counts = _sc_partition(srcp, dstp)
    h = _tc_in(x, W_in)
    for _ in range(T - 1):
        agg = _sc_gather_scatter(h.reshape(HPAD, 2, 128), slists, dlists,
                                 counts)
        h = _tc_step(h, agg.reshape(AGG_R, D), W_msg)
    agg = _sc_gather_scatter(h.reshape(HPAD, 2, 128), slists, dlists, counts)
    mu, lv = _tc_head(h, agg.reshape(AGG_R, D), W_msg, gidp, W_fc1, bf,
                      W_mu, bm, W_lv, bl)
    return (mu, lv)


# speedup vs baseline: 1.1101x; 1.0043x over previous
import functools

import jax
import jax.numpy as jnp
from jax import lax
from jax.experimental import pallas as pl
from jax.experimental.pallas import tpu as pltpu
from jax.experimental.pallas import tpu_sc as plsc

N = 10000
E = 160000
D = 256
H = 512
L = 128
G = 256
T = 3

NB = 400
NBLK = N // NB
HPAD = 26 * NB
HALF_T = 5200
ACC2 = 5248
DUMMY_L = 5216
AGG_B1 = 5600
AGG_R = AGG_B1 + ACC2
ZR = ACC2 // 16
E_P = 163840
SLICE_R = 80
SLOT = 88
RB = 16

_mesh = plsc.VectorSubcoreMesh(core_axis_name="c", subcore_axis_name="s")


@functools.partial(
    pl.kernel,
    out_type=[
        jax.ShapeDtypeStruct((32 * SLOT, 128), jnp.int32),
        jax.ShapeDtypeStruct((32 * SLOT, 128), jnp.int32),
        jax.ShapeDtypeStruct((512,), jnp.int32),
    ],
    mesh=_mesh,
    compiler_params=pltpu.CompilerParams(needs_layout_passes=False),
    scratch_types=[
        pltpu.VMEM((SLICE_R, 128), jnp.int32),
        pltpu.VMEM((SLICE_R, 128), jnp.int32),
        pltpu.VMEM((SLOT, 128), jnp.int32),
        pltpu.VMEM((SLOT, 128), jnp.int32),
        pltpu.VMEM((256,), jnp.int32),
        pltpu.VMEM((256,), jnp.int32),
        pltpu.VMEM((16,), jnp.int32),
    ],
)
def _sc_partition(src_hbm, dst_hbm, slists_hbm, dlists_hbm, counts_hbm,
                  src_st, dst_st, s_list, d_list, srow, drow, cbuf):
    c = lax.axis_index("c")
    s = lax.axis_index("s")
    wid = c * 16 + s
    pltpu.sync_copy(src_hbm.at[pl.ds(s * SLICE_R, SLICE_R)], src_st)
    pltpu.sync_copy(dst_hbm.at[pl.ds(s * SLICE_R, SLICE_R)], dst_st)
    lo = c * HALF_T

    def flush(nrow):
        for k in range(8):
            sl = pl.ds(16 * k, 16)
            s_list[nrow, sl] = srow[sl]
            d_list[nrow, sl] = drow[sl]

    trash = jnp.arange(16, dtype=jnp.int32) + 240
    ones = jnp.ones((16,), jnp.int32)
    zeros = jnp.zeros((16,), jnp.int32)

    def row_loop(r, carry):
        wp, nrow = carry
        for k in range(8):
            sl = pl.ds(16 * k, 16)
            sv = src_st[r, sl]
            dv = dst_st[r, sl]
            t = dv - lo
            m = jnp.logical_and(t >= 0, t < HALF_T)
            mi = jnp.where(m, ones, zeros)
            offs = jnp.where(m, wp + jnp.cumsum(mi) - 1, trash)
            plsc.store_scatter(srow, [offs], sv)
            plsc.store_scatter(drow, [offs], t)
            wp = wp + jnp.sum(mi)
            full = wp >= 128

            @pl.when(full)
            def _():
                flush(nrow)
                srow[pl.ds(0, 16)] = srow[pl.ds(128, 16)]
                drow[pl.ds(0, 16)] = drow[pl.ds(128, 16)]

            wp = jnp.where(full, wp - 128, wp)
            nrow = nrow + full.astype(jnp.int32)
        return (wp, nrow)

    wp, nrow = lax.fori_loop(0, SLICE_R, row_loop,
                             (jnp.int32(0), jnp.int32(0)))

    dummy_s = jnp.zeros((16,), jnp.int32)
    dummy_d = jnp.full((16,), DUMMY_L, jnp.int32)
    lane = jnp.arange(16, dtype=jnp.int32)
    for k in range(8):
        offs = wp + lane + 16 * k
        plsc.store_scatter(srow, [offs], dummy_s)
        plsc.store_scatter(drow, [offs], dummy_d)
    flush(nrow)

    cbuf[pl.ds(0, 16)] = jnp.full((16,), nrow + 1, jnp.int32)
    pltpu.sync_copy(s_list, slists_hbm.at[pl.ds(wid * SLOT, SLOT)])
    pltpu.sync_copy(d_list, dlists_hbm.at[pl.ds(wid * SLOT, SLOT)])
    pltpu.sync_copy(cbuf, counts_hbm.at[pl.ds(wid * 16, 16)])


@functools.partial(
    pl.kernel,
    out_type=jax.ShapeDtypeStruct((AGG_R, 2, 128), jnp.float32),
    mesh=_mesh,
    scratch_types=[
        pltpu.VMEM((RB, 128), jnp.int32),
        pltpu.VMEM((RB, 128), jnp.int32),
        pltpu.VMEM((128, 2, 128), jnp.float32),
        pltpu.VMEM((16,), jnp.int32),
        pltpu.VMEM_SHARED((ACC2, 2, 128), jnp.float32),
        pltpu.SemaphoreType.DMA,
        pltpu.SemaphoreType.DMA,
    ],
)
def _sc_gather_scatter(h_hbm, slists_hbm, dlists_hbm, counts_hbm, agg_hbm,
                       sidx, didx, buf, cbuf, acc, sema, semb):
    c = lax.axis_index("c")
    s = lax.axis_index("s")
    wid = c * 16 + s
    slotr = wid * SLOT

    pltpu.sync_copy(counts_hbm.at[pl.ds(wid * 16, 16)], cbuf)
    nsub = cbuf[...][0]

    pltpu.sync_copy(slists_hbm.at[pl.ds(slotr, RB)], sidx)
    pltpu.sync_copy(dlists_hbm.at[pl.ds(slotr, RB)], didx)

    zero = jnp.zeros((16,), jnp.float32)

    def zrow(i, carry):
        for half in range(2):
            for k in range(128 // 16):
                buf[i, half, pl.ds(k * 16, 16)] = zero
        return carry

    lax.fori_loop(0, 128, zrow, 0)
    base = s * ZR
    for k in range(ZR // 128):
        pltpu.sync_copy(buf, acc.at[pl.ds(base + k * 128, 128)])
    rem = ZR % 128
    if rem:
        pltpu.sync_copy(buf.at[pl.ds(0, rem)],
                        acc.at[pl.ds(base + (ZR // 128) * 128, rem)])
    plsc.subcore_barrier()

    def body(g, carry):
        @pl.when(jnp.logical_and(g > 0, lax.rem(g, RB) == 0))
        def _():
            b = (g // RB) * RB
            pltpu.sync_copy(slists_hbm.at[pl.ds(slotr + b, RB)], sidx)
            pltpu.sync_copy(dlists_hbm.at[pl.ds(slotr + b, RB)], didx)

        r = lax.rem(g, RB)
        pltpu.make_async_copy(h_hbm.at[sidx.at[r, pl.ds(0, 64)]],
                              buf.at[pl.ds(0, 64)], sema).start()
        pltpu.make_async_copy(h_hbm.at[sidx.at[r, pl.ds(64, 64)]],
                              buf.at[pl.ds(64, 64)], semb).start()
        pltpu.make_async_copy(h_hbm.at[sidx.at[r, pl.ds(0, 64)]],
                              buf.at[pl.ds(0, 64)], sema).wait()
        pltpu.make_async_copy(h_hbm.at[sidx.at[r, pl.ds(64, 64)]],
                              buf.at[pl.ds(64, 64)], semb).wait()
        pltpu.sync_copy(buf, acc.at[didx.at[r]], add=True)
        return carry

    lax.fori_loop(0, nsub, body, 0)
    plsc.subcore_barrier()

    outb = c * AGG_B1 + s * ZR
    for k in range(ZR // 128):
        pltpu.sync_copy(acc.at[pl.ds(base + k * 128, 128)],
                        agg_hbm.at[pl.ds(outb + k * 128, 128)])
    if rem:
        pltpu.sync_copy(acc.at[pl.ds(base + (ZR // 128) * 128, rem)],
                        agg_hbm.at[pl.ds(outb + (ZR // 128) * 128, rem)])


def _tc_in_body(x_ref, w_ref, o_ref):
    o_ref[...] = jnp.maximum(
        jnp.dot(x_ref[...], w_ref[...], preferred_element_type=jnp.float32), 0.0)


_tc_in = pl.pallas_call(
    _tc_in_body,
    grid=(NBLK,),
    in_specs=[
        pl.BlockSpec((NB, D), lambda i: (i, 0)),
        pl.BlockSpec((D, D), lambda i: (0, 0)),
    ],
    out_specs=pl.BlockSpec((NB, D), lambda i: (i, 0)),
    out_shape=jax.ShapeDtypeStruct((HPAD, D), jnp.float32),
)


def _tc_step_body(h_ref, a_ref, w_ref, o_ref):
    o_ref[...] = jnp.maximum(
        h_ref[...] + jnp.dot(a_ref[...], w_ref[...],
                             preferred_element_type=jnp.float32), 0.0)


_tc_step = pl.pallas_call(
    _tc_step_body,
    grid=(NBLK,),
    in_specs=[
        pl.BlockSpec((NB, D), lambda i: (i, 0)),
        pl.BlockSpec((NB, D), lambda i: (jnp.where(i < HALF_T // NB, i, i + 1), 0)),
        pl.BlockSpec((D, D), lambda i: (0, 0)),
    ],
    out_specs=pl.BlockSpec((NB, D), lambda i: (i, 0)),
    out_shape=jax.ShapeDtypeStruct((HPAD, D), jnp.float32),
)


def _tc_head_body(h_ref, a_ref, wg_ref, gid_ref, wf_ref, bf_ref, wm_ref,
                  bm_ref, wl_ref, bl_ref, mu_ref, lv_ref, g_acc, c_acc):
    i = pl.program_id(0)

    @pl.when(i == 0)
    def _():
        g_acc[...] = jnp.zeros_like(g_acc)
        c_acc[...] = jnp.zeros_like(c_acc)

    hblk = jnp.maximum(
        h_ref[...] + jnp.dot(a_ref[...], wg_ref[...],
                             preferred_element_type=jnp.float32), 0.0)
    gid = gid_ref[0]
    mt = (lax.broadcasted_iota(jnp.int32, (G, NB), 0) == gid).astype(jnp.float32)
    g_acc[...] += jnp.dot(mt, hblk, preferred_element_type=jnp.float32)
    c_acc[...] += jnp.sum(mt, axis=1, keepdims=True)

    @pl.when(i == NBLK - 1)
    def _():
        cnt = jnp.maximum(c_acc[...], 1.0)
        g = g_acc[...] / cnt
        hh = jnp.maximum(
            jnp.dot(g, wf_ref[...], preferred_element_type=jnp.float32)
            + bf_ref[...], 0.0)
        mu_ref[...] = jnp.dot(hh, wm_ref[...],
                              preferred_element_type=jnp.float32) + bm_ref[...]
        lv_ref[...] = jnp.dot(hh, wl_ref[...],
                              preferred_element_type=jnp.float32) + bl_ref[...]


_tc_head = pl.pallas_call(
    _tc_head_body,
    grid=(NBLK,),
    in_specs=[
        pl.BlockSpec((NB, D), lambda i: (i, 0)),
        pl.BlockSpec((NB, D), lambda i: (jnp.where(i < HALF_T // NB, i, i + 1), 0)),
        pl.BlockSpec((D, D), lambda i: (0, 0)),
        pl.BlockSpec((1, 1, NB), lambda i: (i, 0, 0)),
        pl.BlockSpec((D, H), lambda i: (0, 0)),
        pl.BlockSpec((1, H), lambda i: (0, 0)),
        pl.BlockSpec((H, L), lambda i: (0, 0)),
        pl.BlockSpec((1, L), lambda i: (0, 0)),
        pl.BlockSpec((H, L), lambda i: (0, 0)),
        pl.BlockSpec((1, L), lambda i: (0, 0)),
    ],
    out_specs=[
        pl.BlockSpec((G, L), lambda i: (0, 0)),
        pl.BlockSpec((G, L), lambda i: (0, 0)),
    ],
    out_shape=[
        jax.ShapeDtypeStruct((G, L), jnp.float32),
        jax.ShapeDtypeStruct((G, L), jnp.float32),
    ],
    scratch_shapes=[
        pltpu.VMEM((G, D), jnp.float32),
        pltpu.VMEM((G, 1), jnp.float32),
    ],
)


@jax.jit
def kernel(x, edge_index, graph_ids, W_in, W_msg, W_fc1, b_fc1, W_mu, b_mu,
           W_lv, b_lv):
    src = edge_index[0]
    dst = edge_index[1]
    pad = E_P - E
    srcp = jnp.concatenate([src, jnp.zeros((pad,), jnp.int32)]
                           ).reshape(E_P // 128, 128)
    dstp = jnp.concatenate([dst, jnp.full((pad,), 2 * HALF_T, jnp.int32)]
                           ).reshape(E_P // 128, 128)
    gidp = graph_ids.reshape(NBLK, 1, NB)
    bf = b_fc1.reshape(1, H)
    bm = b_mu.reshape(1, L)
    bl = b_lv.reshape(1, L)

    slists, dlists, counts = _sc_partition(srcp, dstp)
    h = _tc_in(x, W_in)
    for _ in range(T - 1):
        agg = _sc_gather_scatter(h.reshape(HPAD, 2, 128), slists, dlists,
                                 counts)
        h = _tc_step(h, agg.reshape(AGG_R, D), W_msg)
    agg = _sc_gather_scatter(h.reshape(HPAD, 2, 128), slists, dlists, counts)
    mu, lv = _tc_head(h, agg.reshape(AGG_R, D), W_msg, gidp, W_fc1, bf,
                      W_mu, bm, W_lv, bl)
    return (mu, lv)
